# Initial kernel scaffold; baseline (speedup 1.0000x reference)
import jax
import jax.numpy as jnp
from jax.experimental import pallas as pl


def kernel(input, conv1_w, conv1_b, conv2_w, conv2_b, w_gate, W1, b1, W2, b2):
    raise NotImplementedError("write your pallas kernel here")


# speedup vs baseline: 1.5310x; 1.5310x over previous
import jax
import jax.numpy as jnp
from jax.experimental import pallas as pl
from jax.experimental.pallas import tpu as pltpu

B, T, IW, E = 32, 8, 32, 8
D = IW * IW
HID = 2 * D
NM = 5
f32 = jnp.float32
bf16 = jnp.bfloat16


def _bf(a):
    return a.astype(bf16).astype(f32)


def _cv2(v):
    m = jnp.mean(v)
    var = jnp.sum((v - m) ** 2) / (E - 1)
    return var / (m * m + 1e-10)


def _stem_kernel(xf_ref, w1_ref, b1c_ref, w2s_ref, b2c_ref, wg_ref,
                 x_ref, gates_ref, aux_ref, o1_ref, a2_ref):
    xf = _bf(xf_ref[...])
    lane = jax.lax.broadcasted_iota(jnp.int32, (1, 1024), 1)
    hh = lane // IW
    ww = lane % IW
    row = jax.lax.broadcasted_iota(jnp.int32, (B * T, 1), 0)
    dd = row % T

    def shift(a, sd, sh, sw):
        if sh or sw:
            a = jnp.roll(a, -(sh * IW + sw), axis=1)
            cond = jnp.ones((1, 1024), jnp.bool_)
            if sh:
                cond = cond & ((hh + sh >= 0) & (hh + sh < IW))
            if sw:
                cond = cond & ((ww + sw >= 0) & (ww + sw < IW))
            a = jnp.where(cond, a, 0.0)
        if sd:
            a = jnp.roll(a, -sd, axis=0)
            condr = (dd + sd >= 0) & (dd + sd < T)
            a = jnp.where(condr, a, 0.0)
        return a

    o1_ref[...] = jnp.zeros((10, B * T, 1024), f32)
    k = 0
    for kd in range(3):
        for kh in range(3):
            for kw in range(3):
                s = shift(xf, kd - 1, kh - 1, kw - 1)

                def _acc1(c, _, _k=k, _s=s):
                    o1_ref[c] = o1_ref[c] + _s * _bf(w1_ref[c, _k])
                    return 0

                jax.lax.fori_loop(0, 10, _acc1, 0)
                k += 1

    def _relu1(c, _):
        o1_ref[c] = _bf(jnp.maximum(o1_ref[c] + b1c_ref[0, c], 0.0))
        return 0

    jax.lax.fori_loop(0, 10, _relu1, 0)

    a2_ref[...] = jnp.zeros((B * T, 1024), f32)
    for kh in range(3):
        for kw in range(3):
            j = kh * 3 + kw

            def _acc2(c, _, _j=j, _kh=kh, _kw=kw):
                wcol = jnp.zeros((B * T, 1), f32)
                for d in range(T):
                    wcol = jnp.where(dd == d, w2s_ref[c, d * 9 + _j], wcol)
                a2_ref[...] = a2_ref[...] + shift(o1_ref[c], 0, _kh - 1,
                                                  _kw - 1) * wcol
                return 0

            jax.lax.fori_loop(0, 10, _acc2, 0)
    xs = a2_ref[...].reshape(B, T, 1024).sum(axis=1) + b2c_ref[0, 0]
    xs = jnp.maximum(xs, 0.0)
    x_ref[...] = xs

    lg_all = jnp.dot(xs.astype(bf16), wg_ref[...].astype(bf16),
                     preferred_element_type=f32)
    ei = jax.lax.broadcasted_iota(jnp.int32, (B, E), 1)
    aux = jnp.float32(0.0)
    for i in range(NM):
        lg = lg_all[:, i * E:(i + 1) * E]
        m1 = jnp.max(lg, axis=1, keepdims=True)
        idx1 = jnp.min(jnp.where(lg == m1, ei, E), axis=1, keepdims=True)
        oh1 = ei == idx1
        masked = jnp.where(oh1, -jnp.inf, lg)
        m2 = jnp.max(masked, axis=1, keepdims=True)
        idx2 = jnp.min(jnp.where(masked == m2, ei, E), axis=1, keepdims=True)
        oh2 = ei == idx2
        e2 = jnp.exp(m2 - m1)
        g1 = 1.0 / (1.0 + e2)
        g2 = e2 / (1.0 + e2)
        gates = jnp.where(oh1, g1, 0.0) + jnp.where(oh2, g2, 0.0)
        gates_ref[i] = gates
        imp = jnp.sum(gates, axis=0, keepdims=True)
        load = jnp.sum((gates > 0).astype(f32), axis=0, keepdims=True)
        aux = aux + (_cv2(imp) + _cv2(load)) * 1e-2
    aux_ref[0, 0] = aux


def _ffn_kernel(x_ref, w1_ref, b1_ref, w2_ref, b2_ref, p_ref):
    xb = x_ref[...].astype(bf16)
    h = jnp.dot(xb, w1_ref[0, 0].astype(bf16),
                preferred_element_type=f32) + b1_ref[0, 0]
    h = jnp.maximum(h, 0.0).astype(bf16)
    o = jnp.dot(h, w2_ref[0, 0].astype(bf16),
                preferred_element_type=f32) + b2_ref[0, 0]
    m = jnp.max(o, axis=1, keepdims=True)
    p = jnp.exp(o - m)
    p_ref[0, 0] = p / jnp.sum(p, axis=1, keepdims=True)


def _combine_kernel(inp_ref, p_ref, gt_ref, out_ref):
    funcs = []
    for i in range(NM):
        acc = None
        for e in range(E):
            t = p_ref[i, e] * gt_ref[i, e]
            acc = t if acc is None else acc + t
        funcs.append(acc[:, None, :])
    f1, f0, f2, f3, f4 = funcs
    x = inp_ref[...]
    x2 = x * x
    x3 = x2 * x
    x4 = x2 * x2
    arg = x4 * f4 + x3 * f3 + x2 * f2 + x * f1 + f0
    out_ref[...] = 1.0 / (1.0 + jnp.exp(-arg))


def kernel(input, conv1_w, conv1_b, conv2_w, conv2_b, w_gate, W1, b1, W2, b2):
    xf = input.reshape(B * T, 1024)
    w1f = conv1_w.reshape(10, 27)
    b1c = conv1_b.reshape(1, 10)
    w2s = conv2_w.reshape(10, T * 9)
    b2c = conv2_b.reshape(1, 1)
    wgf = jnp.transpose(w_gate, (1, 0, 2)).reshape(1024, NM * E)

    smem = pl.BlockSpec(memory_space=pltpu.SMEM)
    x, gates, aux = pl.pallas_call(
        _stem_kernel,
        in_specs=[pl.BlockSpec(xf.shape, lambda: (0, 0)),
                  smem, smem, smem, smem,
                  pl.BlockSpec(wgf.shape, lambda: (0, 0))],
        out_specs=[pl.BlockSpec((B, 1024), lambda: (0, 0)),
                   pl.BlockSpec((NM, B, E), lambda: (0, 0, 0)),
                   pl.BlockSpec(memory_space=pltpu.SMEM)],
        out_shape=[jax.ShapeDtypeStruct((B, 1024), f32),
                   jax.ShapeDtypeStruct((NM, B, E), f32),
                   jax.ShapeDtypeStruct((1, 1), f32)],
        scratch_shapes=[pltpu.VMEM((10, B * T, 1024), f32),
                        pltpu.VMEM((B * T, 1024), f32)],
    )(xf, w1f, b1c, w2s, b2c, wgf)

    b1r = b1.reshape(NM, E, 1, HID)
    b2r = b2.reshape(NM, E, 1, D)
    p = pl.pallas_call(
        _ffn_kernel,
        grid=(E, NM),
        in_specs=[pl.BlockSpec((B, 1024), lambda e, i: (0, 0)),
                  pl.BlockSpec((1, 1, 1024, HID), lambda e, i: (i, e, 0, 0)),
                  pl.BlockSpec((1, 1, 1, HID), lambda e, i: (i, e, 0, 0)),
                  pl.BlockSpec((1, 1, HID, D), lambda e, i: (i, e, 0, 0)),
                  pl.BlockSpec((1, 1, 1, D), lambda e, i: (i, e, 0, 0))],
        out_specs=pl.BlockSpec((1, 1, B, D), lambda e, i: (i, e, 0, 0)),
        out_shape=jax.ShapeDtypeStruct((NM, E, B, D), f32),
        compiler_params=pltpu.CompilerParams(
            dimension_semantics=("parallel", "parallel")),
    )(x, W1, b1r, W2, b2r)

    gt = jnp.transpose(gates, (0, 2, 1)).reshape(NM, E, B, 1)
    inp3 = input.reshape(B, T, 1024)
    out = pl.pallas_call(
        _combine_kernel,
        in_specs=[pl.BlockSpec(inp3.shape, lambda: (0, 0, 0)),
                  pl.BlockSpec(p.shape, lambda: (0, 0, 0, 0)),
                  pl.BlockSpec(gt.shape, lambda: (0, 0, 0, 0))],
        out_specs=pl.BlockSpec(inp3.shape, lambda: (0, 0, 0)),
        out_shape=jax.ShapeDtypeStruct(inp3.shape, f32),
    )(inp3, p, gt)

    return out.reshape(B, T, 1, IW, IW), aux.reshape(())
